# baseline (speedup 1.0000x reference)
import jax
import jax.numpy as jnp
from jax.experimental import pallas as pl
from jax.experimental.pallas import tpu as pltpu

_WEIGHT = 1e-7
_SIGMA_RGB = 15.0
_SIGMA_XY = 100.0
_SCALE = 0.5

_BR = 512
_BC = 512
_RB = 8


def _prep_body(img_ref, seg_ref, ft_ref, st_ref, hc_ref):
    hs = img_ref.shape[2] // 2
    ws = img_ref.shape[3] // 2
    p_total = hs * ws
    wfull = img_ref.shape[3]
    v4 = img_ref[0].reshape(3, hs, 2, wfull)
    vr = v4[:, :, 0, :]
    vt = jnp.transpose(vr, (0, 2, 1))
    vt4 = vt.reshape(3, ws, 2, hs)
    img_xy = vt4[:, :, 0, :] / _SIGMA_RGB
    kk = seg_ref.shape[1]
    s4 = seg_ref[0].reshape(kk, hs, 2, wfull)
    sr = s4[:, :, 0, :] + s4[:, :, 1, :]
    st = jnp.transpose(sr, (0, 2, 1))
    st4 = st.reshape(kk, ws, 2, hs)
    seg_xy = (st4[:, :, 0, :] + st4[:, :, 1, :]) * 0.25

    k = seg_xy.shape[0]
    for x in range(ws):
        sl = slice(hs * x, hs * (x + 1))
        ft_ref[0, 2:5, sl] = img_xy[:, x, :]
        st_ref[0, 0:k, sl] = seg_xy[:, x, :]

    lane = jax.lax.broadcasted_iota(jnp.int32, (1, p_total), 1)
    sigma_xy_eff = _SIGMA_XY * _SCALE
    ft_ref[0, 0:1, :] = (lane // hs).astype(jnp.float32) / sigma_xy_eff
    ft_ref[0, 1:2, :] = (lane % hs).astype(jnp.float32) / sigma_xy_eff
    ft_ref[0, 5:8, :] = jnp.zeros((3, p_total), jnp.float32)
    st_ref[0, k:8, :] = jnp.zeros((8 - k, p_total), jnp.float32)
    f = ft_ref[0]
    hc_ref[0] = 0.5 * jnp.sum(f * f, axis=0, keepdims=True)


def _tile_body(ftr_ref, ftc_ref, str_ref, stc_ref, hc_ref, o_ref, hr_ref):
    i = pl.program_id(0)
    j = pl.program_id(1)
    r = jax.lax.rem(i, _RB)

    @pl.when(j == 0)
    def _init():
        frt = jnp.transpose(ftr_ref[0])
        hr_ref[...] = 0.5 * jnp.sum(frt * frt, axis=1, keepdims=True)
        o_ref[...] = jnp.zeros_like(o_ref)

    @pl.when(j >= r)
    def _compute():
        ff = jax.lax.dot_general(ftr_ref[0], ftc_ref[0],
                                 (((0,), (0,)), ((), ())),
                                 preferred_element_type=jnp.float32)
        g = jax.lax.dot_general(str_ref[0], stc_ref[0],
                                (((0,), (0,)), ((), ())),
                                preferred_element_type=jnp.float32)
        t = jnp.minimum((ff - hr_ref[...]) - hc_ref[0], 0.0)
        part = jnp.sum(jnp.exp(t) * g, axis=0)[None, None, :]
        wgt = jnp.where(j == r, 1.0, 2.0).astype(jnp.float32)
        o_ref[...] += part * wgt


def kernel(images, segmentations):
    n, k, h, w = segmentations.shape
    hs, ws = h // 2, w // 2
    P = hs * ws

    FT, ST, HC = pl.pallas_call(
        _prep_body,
        out_shape=[
            jax.ShapeDtypeStruct((n, 8, P), jnp.float32),
            jax.ShapeDtypeStruct((n, 8, P), jnp.float32),
            jax.ShapeDtypeStruct((n, 1, P), jnp.float32),
        ],
        grid=(n,),
        in_specs=[
            pl.BlockSpec((1, 3, h, w), lambda b: (b, 0, 0, 0)),
            pl.BlockSpec((1, k, h, w), lambda b: (b, 0, 0, 0)),
        ],
        out_specs=[
            pl.BlockSpec((1, 8, P), lambda b: (b, 0, 0)),
            pl.BlockSpec((1, 8, P), lambda b: (b, 0, 0)),
            pl.BlockSpec((1, 1, P), lambda b: (b, 0, 0)),
        ],
        compiler_params=pltpu.CompilerParams(
            dimension_semantics=("arbitrary",),
        ),
        name="dense_crf_prep",
    )(images, segmentations)

    rb = P // _BR
    cb = P // _BC
    g0 = n * rb

    partials = pl.pallas_call(
        _tile_body,
        out_shape=jax.ShapeDtypeStruct((g0, 1, _BC), jnp.float32),
        grid=(g0, cb),
        in_specs=[
            pl.BlockSpec((1, 8, _BR), lambda i, j: (i // rb, 0, i % rb)),
            pl.BlockSpec((1, 8, _BC), lambda i, j: (i // rb, 0, j)),
            pl.BlockSpec((1, 8, _BR), lambda i, j: (i // rb, 0, i % rb)),
            pl.BlockSpec((1, 8, _BC), lambda i, j: (i // rb, 0, j)),
            pl.BlockSpec((1, 1, _BC), lambda i, j: (i // rb, 0, j)),
        ],
        out_specs=pl.BlockSpec((1, 1, _BC), lambda i, j: (i, 0, 0)),
        scratch_shapes=[pltpu.VMEM((_BR, 1), jnp.float32)],
        compiler_params=pltpu.CompilerParams(
            dimension_semantics=("parallel", "arbitrary"),
        ),
        name="dense_crf_loss",
    )(FT, FT, ST, ST, HC)

    loss = -jnp.sum(partials) / jnp.float32(n)
    return (_WEIGHT * loss).reshape(1)


# speedup vs baseline: 1.2760x; 1.2760x over previous
import jax
import jax.numpy as jnp
from jax.experimental import pallas as pl
from jax.experimental.pallas import tpu as pltpu

_WEIGHT = 1e-7
_SIGMA_RGB = 15.0
_SIGMA_XY = 100.0
_SCALE = 0.5

_BR = 512
_BC = 512
_RB = 8


def _prep_body(img_ref, seg_ref, ft_ref, st_ref, hc_ref):
    hs = img_ref.shape[2] // 2
    ws = img_ref.shape[3] // 2
    p_total = hs * ws
    wfull = img_ref.shape[3]
    v4 = img_ref[0].reshape(3, hs, 2, wfull)
    vr = v4[:, :, 0, :]
    vt = jnp.transpose(vr, (0, 2, 1))
    vt4 = vt.reshape(3, ws, 2, hs)
    img_xy = vt4[:, :, 0, :] / _SIGMA_RGB
    kk = seg_ref.shape[1]
    s4 = seg_ref[0].reshape(kk, hs, 2, wfull)
    sr = s4[:, :, 0, :] + s4[:, :, 1, :]
    st = jnp.transpose(sr, (0, 2, 1))
    st4 = st.reshape(kk, ws, 2, hs)
    seg_xy = (st4[:, :, 0, :] + st4[:, :, 1, :]) * 0.25

    k = seg_xy.shape[0]
    for x in range(ws):
        sl = slice(hs * x, hs * (x + 1))
        ft_ref[0, 2:5, sl] = img_xy[:, x, :]
        st_ref[0, 0:k, sl] = seg_xy[:, x, :]

    lane = jax.lax.broadcasted_iota(jnp.int32, (1, p_total), 1)
    sigma_xy_eff = _SIGMA_XY * _SCALE
    ft_ref[0, 0:1, :] = (lane // hs).astype(jnp.float32) / sigma_xy_eff
    ft_ref[0, 1:2, :] = (lane % hs).astype(jnp.float32) / sigma_xy_eff
    ft_ref[0, 5:8, :] = jnp.zeros((3, p_total), jnp.float32)
    st_ref[0, k:8, :] = jnp.zeros((8 - k, p_total), jnp.float32)
    f = ft_ref[0]
    hc_ref[0] = 0.5 * jnp.sum(f * f, axis=0, keepdims=True)


def _tile_body(ftr_ref, ftc_ref, str_ref, stc_ref, hc_ref, o_ref, hr_ref):
    j = pl.program_id(1)

    @pl.when(j == 0)
    def _init():
        frt = jnp.transpose(ftr_ref[0])
        hr_ref[...] = 0.5 * jnp.sum(frt * frt, axis=1, keepdims=True)
        o_ref[...] = jnp.zeros_like(o_ref)

    ff = jax.lax.dot_general(ftr_ref[0], ftc_ref[0],
                             (((0,), (0,)), ((), ())),
                             preferred_element_type=jnp.float32)
    g = jax.lax.dot_general(str_ref[0], stc_ref[0],
                            (((0,), (0,)), ((), ())),
                            preferred_element_type=jnp.float32)
    t = jnp.minimum((ff - hr_ref[...]) - hc_ref[0], 0.0)
    part = jnp.sum(jnp.exp(t) * g, axis=0)[None, None, :]
    wgt = jnp.where((j == 0) | (j == _RB // 2), 1.0, 2.0).astype(jnp.float32)
    o_ref[...] += part * wgt


def kernel(images, segmentations):
    n, k, h, w = segmentations.shape
    hs, ws = h // 2, w // 2
    P = hs * ws

    FT, ST, HC = pl.pallas_call(
        _prep_body,
        out_shape=[
            jax.ShapeDtypeStruct((n, 8, P), jnp.float32),
            jax.ShapeDtypeStruct((n, 8, P), jnp.float32),
            jax.ShapeDtypeStruct((n, 1, P), jnp.float32),
        ],
        grid=(n,),
        in_specs=[
            pl.BlockSpec((1, 3, h, w), lambda b: (b, 0, 0, 0)),
            pl.BlockSpec((1, k, h, w), lambda b: (b, 0, 0, 0)),
        ],
        out_specs=[
            pl.BlockSpec((1, 8, P), lambda b: (b, 0, 0)),
            pl.BlockSpec((1, 8, P), lambda b: (b, 0, 0)),
            pl.BlockSpec((1, 1, P), lambda b: (b, 0, 0)),
        ],
        compiler_params=pltpu.CompilerParams(
            dimension_semantics=("arbitrary",),
        ),
        name="dense_crf_prep",
    )(images, segmentations)

    rb = P // _BR
    g0 = n * rb
    nd = _RB // 2 + 1

    partials = pl.pallas_call(
        _tile_body,
        out_shape=jax.ShapeDtypeStruct((g0, 1, _BC), jnp.float32),
        grid=(g0, nd),
        in_specs=[
            pl.BlockSpec((1, 8, _BR), lambda i, j: (i // rb, 0, i % rb)),
            pl.BlockSpec((1, 8, _BC), lambda i, j: (i // rb, 0, (i + j) % rb)),
            pl.BlockSpec((1, 8, _BR), lambda i, j: (i // rb, 0, i % rb)),
            pl.BlockSpec((1, 8, _BC), lambda i, j: (i // rb, 0, (i + j) % rb)),
            pl.BlockSpec((1, 1, _BC), lambda i, j: (i // rb, 0, (i + j) % rb)),
        ],
        out_specs=pl.BlockSpec((1, 1, _BC), lambda i, j: (i, 0, 0)),
        scratch_shapes=[pltpu.VMEM((_BR, 1), jnp.float32)],
        compiler_params=pltpu.CompilerParams(
            dimension_semantics=("parallel", "arbitrary"),
        ),
        name="dense_crf_loss",
    )(FT, FT, ST, ST, HC)

    loss = -jnp.sum(partials) / jnp.float32(n)
    return (_WEIGHT * loss).reshape(1)


# speedup vs baseline: 2.6406x; 2.0695x over previous
import jax
import jax.numpy as jnp
from jax.experimental import pallas as pl
from jax.experimental.pallas import tpu as pltpu

_WEIGHT = 1e-7
_SIGMA_RGB = 15.0
_SIGMA_XY = 100.0
_SCALE = 0.5

_BR = 512
_BC = 512
_RB = 8


def _prep_body(img_ref, seg_ref, ft_ref, st_ref, hc_ref):
    hs = img_ref.shape[2] // 2
    ws = img_ref.shape[3] // 2
    p_total = hs * ws
    wfull = img_ref.shape[3]
    v4 = img_ref[0].reshape(3, hs, 2, wfull)
    vr = v4[:, :, 0, :]
    vt = jnp.transpose(vr, (0, 2, 1))
    vt4 = vt.reshape(3, ws, 2, hs)
    img_xy = vt4[:, :, 0, :] / _SIGMA_RGB
    kk = seg_ref.shape[1]
    s4 = seg_ref[0].reshape(kk, hs, 2, wfull)
    sr = s4[:, :, 0, :] + s4[:, :, 1, :]
    st = jnp.transpose(sr, (0, 2, 1))
    st4 = st.reshape(kk, ws, 2, hs)
    seg_xy = (st4[:, :, 0, :] + st4[:, :, 1, :]) * 0.25

    k = seg_xy.shape[0]
    for x in range(ws):
        sl = slice(hs * x, hs * (x + 1))
        ft_ref[0, 2:5, sl] = img_xy[:, x, :]
        st_ref[0, 0:k, sl] = seg_xy[:, x, :]

    lane = jax.lax.broadcasted_iota(jnp.int32, (1, p_total), 1)
    sigma_xy_eff = _SIGMA_XY * _SCALE
    ft_ref[0, 0:1, :] = (lane // hs).astype(jnp.float32) / sigma_xy_eff
    ft_ref[0, 1:2, :] = (lane % hs).astype(jnp.float32) / sigma_xy_eff
    ft_ref[0, 5:8, :] = jnp.zeros((3, p_total), jnp.float32)
    st_ref[0, k:8, :] = jnp.zeros((8 - k, p_total), jnp.float32)
    f = ft_ref[0]
    hc_ref[0] = 0.5 * jnp.sum(f * f, axis=0, keepdims=True)


def _tile_body(ftr_ref, ftf_ref, str_ref, stf_ref, hcf_ref, o_ref):
    i = pl.program_id(0)
    r = jax.lax.rem(i, _RB)

    frt = jnp.transpose(ftr_ref[0])
    hr = 0.5 * jnp.sum(frt * frt, axis=1, keepdims=True)

    acc = jnp.zeros((1, _BC), jnp.float32)
    for d in range(_RB // 2 + 1):
        c = jax.lax.rem(r + d, _RB)
        off = pl.multiple_of(c * _BC, _BC)
        ftc = ftf_ref[0, :, pl.ds(off, _BC)]
        stc = stf_ref[0, :, pl.ds(off, _BC)]
        hcc = hcf_ref[0, :, pl.ds(off, _BC)]
        ff = jax.lax.dot_general(ftr_ref[0], ftc,
                                 (((0,), (0,)), ((), ())),
                                 preferred_element_type=jnp.float32)
        g = jax.lax.dot_general(str_ref[0], stc,
                                (((0,), (0,)), ((), ())),
                                preferred_element_type=jnp.float32)
        t = jnp.minimum((ff - hr) - hcc, 0.0)
        part = jnp.sum(jnp.exp(t) * g, axis=0, keepdims=True)
        wgt = 1.0 if (d == 0 or d == _RB // 2) else 2.0
        acc = acc + part * wgt
    o_ref[...] = acc[None]


def kernel(images, segmentations):
    n, k, h, w = segmentations.shape
    hs, ws = h // 2, w // 2
    P = hs * ws

    FT, ST, HC = pl.pallas_call(
        _prep_body,
        out_shape=[
            jax.ShapeDtypeStruct((n, 8, P), jnp.float32),
            jax.ShapeDtypeStruct((n, 8, P), jnp.float32),
            jax.ShapeDtypeStruct((n, 1, P), jnp.float32),
        ],
        grid=(n,),
        in_specs=[
            pl.BlockSpec((1, 3, h, w), lambda b: (b, 0, 0, 0)),
            pl.BlockSpec((1, k, h, w), lambda b: (b, 0, 0, 0)),
        ],
        out_specs=[
            pl.BlockSpec((1, 8, P), lambda b: (b, 0, 0)),
            pl.BlockSpec((1, 8, P), lambda b: (b, 0, 0)),
            pl.BlockSpec((1, 1, P), lambda b: (b, 0, 0)),
        ],
        compiler_params=pltpu.CompilerParams(
            dimension_semantics=("arbitrary",),
        ),
        name="dense_crf_prep",
    )(images, segmentations)

    rb = P // _BR
    g0 = n * rb

    partials = pl.pallas_call(
        _tile_body,
        out_shape=jax.ShapeDtypeStruct((g0, 1, _BC), jnp.float32),
        grid=(g0,),
        in_specs=[
            pl.BlockSpec((1, 8, _BR), lambda i: (i // rb, 0, i % rb)),
            pl.BlockSpec((1, 8, P), lambda i: (i // rb, 0, 0)),
            pl.BlockSpec((1, 8, _BR), lambda i: (i // rb, 0, i % rb)),
            pl.BlockSpec((1, 8, P), lambda i: (i // rb, 0, 0)),
            pl.BlockSpec((1, 1, P), lambda i: (i // rb, 0, 0)),
        ],
        out_specs=pl.BlockSpec((1, 1, _BC), lambda i: (i, 0, 0)),
        compiler_params=pltpu.CompilerParams(
            dimension_semantics=("parallel",),
        ),
        name="dense_crf_loss",
    )(FT, FT, ST, ST, HC)

    loss = -jnp.sum(partials) / jnp.float32(n)
    return (_WEIGHT * loss).reshape(1)


# speedup vs baseline: 2.6547x; 1.0053x over previous
import jax
import jax.numpy as jnp
from jax.experimental import pallas as pl
from jax.experimental.pallas import tpu as pltpu

_WEIGHT = 1e-7
_SIGMA_RGB = 15.0
_SIGMA_XY = 100.0
_SCALE = 0.5

_BR = 512
_BC = 512
_RB = 8


def _prep_body(img_ref, seg_ref, ft_ref, st_ref, hc_ref):
    hs = img_ref.shape[2] // 2
    ws = img_ref.shape[3] // 2
    p_total = hs * ws
    wfull = img_ref.shape[3]
    v4 = img_ref[0].reshape(3, hs, 2, wfull)
    vr = v4[:, :, 0, :]
    vt = jnp.transpose(vr, (0, 2, 1))
    vt4 = vt.reshape(3, ws, 2, hs)
    img_xy = vt4[:, :, 0, :] / _SIGMA_RGB
    kk = seg_ref.shape[1]
    s4 = seg_ref[0].reshape(kk, hs, 2, wfull)
    sr = s4[:, :, 0, :] + s4[:, :, 1, :]
    st = jnp.transpose(sr, (0, 2, 1))
    st4 = st.reshape(kk, ws, 2, hs)
    seg_xy = (st4[:, :, 0, :] + st4[:, :, 1, :]) * 0.25

    k = seg_xy.shape[0]
    for x in range(ws):
        sl = slice(hs * x, hs * (x + 1))
        ft_ref[0, 2:5, sl] = img_xy[:, x, :]
        st_ref[0, 0:k, sl] = seg_xy[:, x, :]

    lane = jax.lax.broadcasted_iota(jnp.int32, (1, p_total), 1)
    sigma_xy_eff = _SIGMA_XY * _SCALE
    ft_ref[0, 0:1, :] = (lane // hs).astype(jnp.float32) / sigma_xy_eff
    ft_ref[0, 1:2, :] = (lane % hs).astype(jnp.float32) / sigma_xy_eff
    ft_ref[0, 5:8, :] = jnp.zeros((3, p_total), jnp.float32)
    st_ref[0, k:8, :] = jnp.zeros((8 - k, p_total), jnp.float32)
    f = ft_ref[0]
    hc_ref[0] = 0.5 * jnp.sum(f * f, axis=0, keepdims=True)


def _tile_body(ftr_ref, ftf_ref, str_ref, stf_ref, hcf_ref, o_ref):
    i = pl.program_id(0)
    r = jax.lax.rem(i, _RB)

    frt = jnp.transpose(ftr_ref[0])
    hr = 0.5 * jnp.sum(frt * frt, axis=1, keepdims=True)

    ftr16 = ftr_ref[0].astype(jnp.bfloat16)
    str16 = str_ref[0].astype(jnp.bfloat16)

    acc = jnp.zeros((1, _BC), jnp.float32)
    for d in range(_RB // 2 + 1):
        c = jax.lax.rem(r + d, _RB)
        off = pl.multiple_of(c * _BC, _BC)
        ftc = ftf_ref[0, :, pl.ds(off, _BC)].astype(jnp.bfloat16)
        stc = stf_ref[0, :, pl.ds(off, _BC)].astype(jnp.bfloat16)
        hcc = hcf_ref[0, :, pl.ds(off, _BC)]
        ff = jax.lax.dot_general(ftr16, ftc,
                                 (((0,), (0,)), ((), ())),
                                 preferred_element_type=jnp.float32)
        g = jax.lax.dot_general(str16, stc,
                                (((0,), (0,)), ((), ())),
                                preferred_element_type=jnp.float32)
        t = jnp.minimum((ff - hr) - hcc, 0.0)
        part = jnp.sum(jnp.exp(t) * g, axis=0, keepdims=True)
        wgt = 1.0 if (d == 0 or d == _RB // 2) else 2.0
        acc = acc + part * wgt
    o_ref[...] = acc[None]


def kernel(images, segmentations):
    n, k, h, w = segmentations.shape
    hs, ws = h // 2, w // 2
    P = hs * ws

    FT, ST, HC = pl.pallas_call(
        _prep_body,
        out_shape=[
            jax.ShapeDtypeStruct((n, 8, P), jnp.float32),
            jax.ShapeDtypeStruct((n, 8, P), jnp.float32),
            jax.ShapeDtypeStruct((n, 1, P), jnp.float32),
        ],
        grid=(n,),
        in_specs=[
            pl.BlockSpec((1, 3, h, w), lambda b: (b, 0, 0, 0)),
            pl.BlockSpec((1, k, h, w), lambda b: (b, 0, 0, 0)),
        ],
        out_specs=[
            pl.BlockSpec((1, 8, P), lambda b: (b, 0, 0)),
            pl.BlockSpec((1, 8, P), lambda b: (b, 0, 0)),
            pl.BlockSpec((1, 1, P), lambda b: (b, 0, 0)),
        ],
        compiler_params=pltpu.CompilerParams(
            dimension_semantics=("arbitrary",),
        ),
        name="dense_crf_prep",
    )(images, segmentations)

    rb = P // _BR
    g0 = n * rb

    partials = pl.pallas_call(
        _tile_body,
        out_shape=jax.ShapeDtypeStruct((g0, 1, _BC), jnp.float32),
        grid=(g0,),
        in_specs=[
            pl.BlockSpec((1, 8, _BR), lambda i: (i // rb, 0, i % rb)),
            pl.BlockSpec((1, 8, P), lambda i: (i // rb, 0, 0)),
            pl.BlockSpec((1, 8, _BR), lambda i: (i // rb, 0, i % rb)),
            pl.BlockSpec((1, 8, P), lambda i: (i // rb, 0, 0)),
            pl.BlockSpec((1, 1, P), lambda i: (i // rb, 0, 0)),
        ],
        out_specs=pl.BlockSpec((1, 1, _BC), lambda i: (i, 0, 0)),
        compiler_params=pltpu.CompilerParams(
            dimension_semantics=("parallel",),
        ),
        name="dense_crf_loss",
    )(FT, FT, ST, ST, HC)

    loss = -jnp.sum(partials) / jnp.float32(n)
    return (_WEIGHT * loss).reshape(1)


# speedup vs baseline: 3.1049x; 1.1696x over previous
import functools

import jax
import jax.numpy as jnp
from jax.experimental import pallas as pl
from jax.experimental.pallas import tpu as pltpu

_WEIGHT = 1e-7
_SIGMA_RGB = 15.0
_SIGMA_XY = 100.0
_SCALE = 0.5

_BR = 512
_BC = 512
_RB = 8


def _prep_body(img_ref, seg_ref, ft_ref, st_ref, hc_ref):
    hs = img_ref.shape[2] // 2
    ws = img_ref.shape[3] // 2
    p_total = hs * ws
    wfull = img_ref.shape[3]
    v4 = img_ref[0].reshape(3, hs, 2, wfull)
    vr = v4[:, :, 0, :]
    vt = jnp.transpose(vr, (0, 2, 1))
    vt4 = vt.reshape(3, ws, 2, hs)
    img_xy = vt4[:, :, 0, :] / _SIGMA_RGB
    kk = seg_ref.shape[1]
    s4 = seg_ref[0].reshape(kk, hs, 2, wfull)
    sr = s4[:, :, 0, :] + s4[:, :, 1, :]
    st = jnp.transpose(sr, (0, 2, 1))
    st4 = st.reshape(kk, ws, 2, hs)
    seg_xy = (st4[:, :, 0, :] + st4[:, :, 1, :]) * 0.25

    k = seg_xy.shape[0]
    for x in range(ws):
        sl = slice(hs * x, hs * (x + 1))
        ft_ref[0, 2:5, sl] = img_xy[:, x, :]
        st_ref[0, 0:k, sl] = seg_xy[:, x, :]

    lane = jax.lax.broadcasted_iota(jnp.int32, (1, p_total), 1)
    sigma_xy_eff = _SIGMA_XY * _SCALE
    ft_ref[0, 0:1, :] = (lane // hs).astype(jnp.float32) / sigma_xy_eff
    ft_ref[0, 1:2, :] = (lane % hs).astype(jnp.float32) / sigma_xy_eff
    ft_ref[0, 5:8, :] = jnp.zeros((3, p_total), jnp.float32)
    st_ref[0, k:8, :] = jnp.zeros((8 - k, p_total), jnp.float32)
    f = ft_ref[0]
    hc_ref[0] = 0.5 * jnp.sum(f * f, axis=0, keepdims=True)


def _tile_body(ftr_ref, ftf_ref, str_ref, stf_ref, hcf_ref, o_ref, acc_ref,
               *, g0, inv_scale):
    i = pl.program_id(0)
    r = jax.lax.rem(i, _RB)

    frt = jnp.transpose(ftr_ref[0])
    hr = 0.5 * jnp.sum(frt * frt, axis=1, keepdims=True)

    ftr16 = ftr_ref[0].astype(jnp.bfloat16)
    str16 = str_ref[0].astype(jnp.bfloat16)

    nd = _RB // 2 + 1
    ws = []
    for d in range(nd):
        c = jax.lax.rem(r + d, _RB)
        off = pl.multiple_of(c * _BC, _BC)
        ftc = ftf_ref[0, :, pl.ds(off, _BC)].astype(jnp.bfloat16)
        ff = jax.lax.dot_general(ftr16, ftc,
                                 (((0,), (0,)), ((), ())),
                                 preferred_element_type=jnp.float32)
        hcc = hcf_ref[0, :, pl.ds(off, _BC)]
        t = jnp.minimum((ff - hr) - hcc, 0.0)
        ws.append(jnp.exp(t))

    acc8 = jnp.zeros((8, _BC), jnp.float32)
    for d in range(nd):
        c = jax.lax.rem(r + d, _RB)
        off = pl.multiple_of(c * _BC, _BC)
        stcf = stf_ref[0, :, pl.ds(off, _BC)]
        z = jax.lax.dot_general(str_ref[0], ws[d], (((1,), (0,)), ((), ())),
                                preferred_element_type=jnp.float32)
        wgt = 1.0 if (d == 0 or d == _RB // 2) else 2.0
        acc8 = acc8 + (z * stcf) * wgt

    @pl.when(i == 0)
    def _init():
        acc_ref[...] = jnp.zeros_like(acc_ref)

    acc_ref[...] += acc8

    @pl.when(i == g0 - 1)
    def _fin():
        o_ref[...] = jnp.sum(acc_ref[...]).reshape(1, 1) * inv_scale


def kernel(images, segmentations):
    n, k, h, w = segmentations.shape
    hs, ws = h // 2, w // 2
    P = hs * ws

    FT, ST, HC = pl.pallas_call(
        _prep_body,
        out_shape=[
            jax.ShapeDtypeStruct((n, 8, P), jnp.float32),
            jax.ShapeDtypeStruct((n, 8, P), jnp.float32),
            jax.ShapeDtypeStruct((n, 1, P), jnp.float32),
        ],
        grid=(n,),
        in_specs=[
            pl.BlockSpec((1, 3, h, w), lambda b: (b, 0, 0, 0)),
            pl.BlockSpec((1, k, h, w), lambda b: (b, 0, 0, 0)),
        ],
        out_specs=[
            pl.BlockSpec((1, 8, P), lambda b: (b, 0, 0)),
            pl.BlockSpec((1, 8, P), lambda b: (b, 0, 0)),
            pl.BlockSpec((1, 1, P), lambda b: (b, 0, 0)),
        ],
        compiler_params=pltpu.CompilerParams(
            dimension_semantics=("arbitrary",),
        ),
        name="dense_crf_prep",
    )(images, segmentations)

    rb = P // _BR
    g0 = n * rb

    body = functools.partial(_tile_body, g0=g0,
                             inv_scale=float(-_WEIGHT / n))
    out = pl.pallas_call(
        body,
        out_shape=jax.ShapeDtypeStruct((1, 1), jnp.float32),
        grid=(g0,),
        in_specs=[
            pl.BlockSpec((1, 8, _BR), lambda i: (i // rb, 0, i % rb)),
            pl.BlockSpec((1, 8, P), lambda i: (i // rb, 0, 0)),
            pl.BlockSpec((1, 8, _BR), lambda i: (i // rb, 0, i % rb)),
            pl.BlockSpec((1, 8, P), lambda i: (i // rb, 0, 0)),
            pl.BlockSpec((1, 1, P), lambda i: (i // rb, 0, 0)),
        ],
        out_specs=pl.BlockSpec((1, 1), lambda i: (0, 0)),
        scratch_shapes=[pltpu.VMEM((8, _BC), jnp.float32)],
        compiler_params=pltpu.CompilerParams(
            dimension_semantics=("arbitrary",),
        ),
        name="dense_crf_loss",
    )(FT, FT, ST, ST, HC)

    return out.reshape(1)
